# baseline (speedup 1.0000x reference)
import functools

import jax
import jax.numpy as jnp
from jax import lax
from jax.experimental import pallas as pl
from jax.experimental.pallas import tpu as pltpu
from jax.experimental.pallas import tpu_sc as plsc

_NUM_BATCH = 16
_NUM_CHAIN = 128
_NSEG = _NUM_BATCH + _NUM_CHAIN
_NPAIR = _NUM_BATCH * _NUM_CHAIN


def _pick_block(n):
    for r in (3200, 6400, 1600, 800, 400, 320, 160, 80, 40, 16, 8):
        if n % r == 0:
            return r
    return n


def _onehot_bf16(b, c, r):
    seg_iota = lax.broadcasted_iota(jnp.int32, (r, _NSEG), 1)
    return ((b[:, None] == seg_iota).astype(jnp.bfloat16)
            + (c[:, None] == (seg_iota - _NUM_BATCH)).astype(jnp.bfloat16))


def _reduce_body(local_ref, batch_ref, chain_ref, mask_ref, sc_ref):
    @pl.when(pl.program_id(0) == 0)
    def _init():
        sc_ref[...] = jnp.zeros_like(sc_ref)

    x = local_ref[...]
    b = batch_ref[0, 0, :]
    c = chain_ref[0, 0, :]
    m = mask_ref[0, 0, :]
    r, d = x.shape
    onehot = _onehot_bf16(b, c, r) * m[:, None].astype(jnp.bfloat16)
    aug = jnp.concatenate(
        [x.astype(jnp.bfloat16), jnp.ones((r, d), jnp.bfloat16)], axis=1)
    sc_ref[...] += lax.dot_general(
        onehot, aug, (((0,), (0,)), ((), ())),
        preferred_element_type=jnp.float32)


def _middle_body(sc_ref, w1_ref, b1_ref, w2_ref, b2_ref, t_ref):
    d = t_ref.shape[1]
    s = sc_ref[:, :d]
    cnt = sc_ref[:, d:]
    mean = s / jnp.maximum(cnt, 1e-6)
    u = lax.dot_general(
        mean, w1_ref[...], (((1,), (0,)), ((), ())),
        preferred_element_type=jnp.float32,
        precision=lax.Precision.HIGHEST) + b1_ref[0, :]
    g = jax.nn.gelu(u)
    a = lax.dot_general(
        g[:_NUM_BATCH], w2_ref[: u.shape[1]],
        (((1,), (0,)), ((), ())),
        preferred_element_type=jnp.float32,
        precision=lax.Precision.HIGHEST)
    bb = lax.dot_general(
        g[_NUM_BATCH:], w2_ref[u.shape[1]:],
        (((1,), (0,)), ((), ())),
        preferred_element_type=jnp.float32,
        precision=lax.Precision.HIGHEST)
    pair = (a + b2_ref[0, :])[:, None, :] + bb[None, :, :]
    t_ref[...] = pair.reshape(_NPAIR, d)


_CHUNK = 80


def _sc_broadcast(n, d):
    info = plsc.get_sparse_core_info()
    nw = info.num_cores * info.num_subcores
    rows_w = n // nw
    assert n % nw == 0 and rows_w % _CHUNK == 0
    nchunk = rows_w // _CHUNK
    mesh = plsc.VectorSubcoreMesh(core_axis_name="c", subcore_axis_name="s")

    @functools.partial(
        pl.kernel, mesh=mesh,
        out_type=jax.ShapeDtypeStruct((n, d), jnp.float32),
        scratch_types=[
            pltpu.VMEM((rows_w,), jnp.int32),
            pltpu.VMEM((rows_w,), jnp.int32),
            pltpu.VMEM((_CHUNK,), jnp.int32),
            pltpu.VMEM((_CHUNK, d), jnp.float32),
            pltpu.SemaphoreType.DMA,
        ],
    )
    def k(batch_hbm, chain_hbm, t_hbm, out_hbm, b_v, c_v, p_v, rows_v, sem):
        wid = lax.axis_index("s") * info.num_cores + lax.axis_index("c")
        base = wid * rows_w
        pltpu.sync_copy(batch_hbm.at[pl.ds(base, rows_w)], b_v)
        pltpu.sync_copy(chain_hbm.at[pl.ds(base, rows_w)], c_v)

        def chunk(i, _):
            off = i * _CHUNK
            for j in range(_CHUNK // 16):
                sl = pl.ds(off + j * 16, 16)
                p_v[pl.ds(j * 16, 16)] = (
                    b_v[sl] * _NUM_CHAIN + c_v[sl])
            pltpu.async_copy(t_hbm.at[p_v], rows_v, sem).wait()
            pltpu.sync_copy(rows_v, out_hbm.at[pl.ds(base + off, _CHUNK)])
            return _

        lax.fori_loop(0, nchunk, chunk, None)

    return k


@jax.jit
def kernel(local, chain, batch, mask, W1, b1, W2, b2):
    n, d = local.shape
    r = _pick_block(n)
    gb = n // r
    batch_i = batch.astype(jnp.int32)
    chain_i = chain.astype(jnp.int32)
    batch3 = batch_i.reshape(gb, 1, r)
    chain3 = chain_i.reshape(gb, 1, r)
    mask3 = mask.astype(jnp.float32).reshape(gb, 1, r)

    sc = pl.pallas_call(
        _reduce_body,
        grid=(gb,),
        in_specs=[
            pl.BlockSpec((r, d), lambda i: (i, 0)),
            pl.BlockSpec((1, 1, r), lambda i: (i, 0, 0)),
            pl.BlockSpec((1, 1, r), lambda i: (i, 0, 0)),
            pl.BlockSpec((1, 1, r), lambda i: (i, 0, 0)),
        ],
        out_specs=pl.BlockSpec((_NSEG, 2 * d), lambda i: (0, 0)),
        out_shape=jax.ShapeDtypeStruct((_NSEG, 2 * d), jnp.float32),
    )(local, batch3, chain3, mask3)

    t = pl.pallas_call(
        _middle_body,
        out_shape=jax.ShapeDtypeStruct((_NPAIR, d), jnp.float32),
    )(sc, W1, b1.reshape(1, -1), W2, b2.reshape(1, -1))

    out = _sc_broadcast(n, d)(batch_i, chain_i, t)
    return out


# speedup vs baseline: 1.0600x; 1.0600x over previous
import functools

import jax
import jax.numpy as jnp
from jax import lax
from jax.experimental import pallas as pl
from jax.experimental.pallas import tpu as pltpu
from jax.experimental.pallas import tpu_sc as plsc

_NUM_BATCH = 16
_NUM_CHAIN = 128
_NSEG = _NUM_BATCH + _NUM_CHAIN
_NPAIR = _NUM_BATCH * _NUM_CHAIN


def _pick_block(n):
    for r in (3200, 6400, 1600, 800, 400, 320, 160, 80, 40, 16, 8):
        if n % r == 0:
            return r
    return n


def _onehot_bf16(b, c, r):
    seg_iota = lax.broadcasted_iota(jnp.int32, (r, _NSEG), 1)
    return ((b[:, None] == seg_iota).astype(jnp.bfloat16)
            + (c[:, None] == (seg_iota - _NUM_BATCH)).astype(jnp.bfloat16))


def _reduce_body(local_ref, batch_ref, chain_ref, mask_ref, sc_ref):
    @pl.when(pl.program_id(0) == 0)
    def _init():
        sc_ref[...] = jnp.zeros_like(sc_ref)

    x = local_ref[...]
    b = batch_ref[0, 0, :]
    c = chain_ref[0, 0, :]
    m = mask_ref[0, 0, :]
    r, d = x.shape
    onehot = _onehot_bf16(b, c, r) * m[:, None].astype(jnp.bfloat16)
    aug = jnp.concatenate(
        [x.astype(jnp.bfloat16), jnp.ones((r, d), jnp.bfloat16)], axis=1)
    sc_ref[...] += lax.dot_general(
        onehot, aug, (((0,), (0,)), ((), ())),
        preferred_element_type=jnp.float32)


def _middle_body(sc_ref, w1_ref, b1_ref, w2_ref, b2_ref, t_ref):
    d = t_ref.shape[1]
    s = sc_ref[:, :d]
    cnt = sc_ref[:, d:]
    mean = s / jnp.maximum(cnt, 1e-6)
    u = lax.dot_general(
        mean, w1_ref[...], (((1,), (0,)), ((), ())),
        preferred_element_type=jnp.float32,
        precision=lax.Precision.HIGHEST) + b1_ref[0, :]
    g = jax.nn.gelu(u)
    a = lax.dot_general(
        g[:_NUM_BATCH], w2_ref[: u.shape[1]],
        (((1,), (0,)), ((), ())),
        preferred_element_type=jnp.float32,
        precision=lax.Precision.HIGHEST)
    bb = lax.dot_general(
        g[_NUM_BATCH:], w2_ref[u.shape[1]:],
        (((1,), (0,)), ((), ())),
        preferred_element_type=jnp.float32,
        precision=lax.Precision.HIGHEST)
    pair = (a + b2_ref[0, :])[:, None, :] + bb[None, :, :]
    t_ref[...] = pair.reshape(_NPAIR, d)


_CHUNK = 80
_NBUF = 5


def _sc_broadcast(n, d):
    info = plsc.get_sparse_core_info()
    nw = info.num_cores * info.num_subcores
    rows_w = n // nw
    assert n % nw == 0 and rows_w % _CHUNK == 0
    nchunk = rows_w // _CHUNK
    assert nchunk % _NBUF == 0
    nround = nchunk // _NBUF
    mesh = plsc.VectorSubcoreMesh(core_axis_name="c", subcore_axis_name="s")

    scratch = (
        [pltpu.VMEM((rows_w,), jnp.int32)] * 2
        + [pltpu.VMEM((_CHUNK,), jnp.int32)] * _NBUF
        + [pltpu.VMEM((_CHUNK, d), jnp.float32)] * _NBUF
        + [pltpu.SemaphoreType.DMA] * (2 * _NBUF)
    )

    @functools.partial(
        pl.kernel, mesh=mesh,
        out_type=jax.ShapeDtypeStruct((n, d), jnp.float32),
        scratch_types=scratch,
    )
    def k(batch_hbm, chain_hbm, t_hbm, out_hbm, *scr):
        b_v, c_v = scr[0], scr[1]
        p_v = scr[2:2 + _NBUF]
        rows_v = scr[2 + _NBUF:2 + 2 * _NBUF]
        sem_g = scr[2 + 2 * _NBUF:2 + 3 * _NBUF]
        sem_w = scr[2 + 3 * _NBUF:2 + 4 * _NBUF]
        wid = lax.axis_index("s") * info.num_cores + lax.axis_index("c")
        base = wid * rows_w
        pltpu.sync_copy(batch_hbm.at[pl.ds(base, rows_w)], b_v)
        pltpu.sync_copy(chain_hbm.at[pl.ds(base, rows_w)], c_v)

        def gather(slot, chunk_idx):
            off = chunk_idx * _CHUNK
            for j in range(_CHUNK // 16):
                sl = pl.ds(off + j * 16, 16)
                p_v[slot][pl.ds(j * 16, 16)] = (
                    b_v[sl] * _NUM_CHAIN + c_v[sl])
            pltpu.async_copy(t_hbm.at[p_v[slot]], rows_v[slot], sem_g[slot])

        for s in range(_NBUF):
            gather(s, s)

        def round_body(g, _):
            for s in range(_NBUF):
                ck = g * _NBUF + s
                pltpu.make_async_copy(
                    t_hbm.at[p_v[s]], rows_v[s], sem_g[s]).wait()
                pltpu.async_copy(
                    rows_v[s], out_hbm.at[pl.ds(base + ck * _CHUNK, _CHUNK)],
                    sem_w[s])
            for s in range(_NBUF):
                pltpu.make_async_copy(
                    rows_v[s],
                    out_hbm.at[pl.ds(base + (g * _NBUF + s) * _CHUNK, _CHUNK)],
                    sem_w[s]).wait()
                gather(s, (g + 1) * _NBUF + s)
            return _

        lax.fori_loop(0, nround - 1, round_body, None)

        g_last = nround - 1
        for s in range(_NBUF):
            ck = g_last * _NBUF + s
            pltpu.make_async_copy(
                t_hbm.at[p_v[s]], rows_v[s], sem_g[s]).wait()
            pltpu.async_copy(
                rows_v[s], out_hbm.at[pl.ds(base + ck * _CHUNK, _CHUNK)],
                sem_w[s])
        for s in range(_NBUF):
            pltpu.make_async_copy(
                rows_v[s],
                out_hbm.at[pl.ds(base + (g_last * _NBUF + s) * _CHUNK, _CHUNK)],
                sem_w[s]).wait()

    return k


@jax.jit
def kernel(local, chain, batch, mask, W1, b1, W2, b2):
    n, d = local.shape
    r = _pick_block(n)
    gb = n // r
    batch_i = batch.astype(jnp.int32)
    chain_i = chain.astype(jnp.int32)
    batch3 = batch_i.reshape(gb, 1, r)
    chain3 = chain_i.reshape(gb, 1, r)
    mask3 = mask.astype(jnp.float32).reshape(gb, 1, r)

    sc = pl.pallas_call(
        _reduce_body,
        grid=(gb,),
        in_specs=[
            pl.BlockSpec((r, d), lambda i: (i, 0)),
            pl.BlockSpec((1, 1, r), lambda i: (i, 0, 0)),
            pl.BlockSpec((1, 1, r), lambda i: (i, 0, 0)),
            pl.BlockSpec((1, 1, r), lambda i: (i, 0, 0)),
        ],
        out_specs=pl.BlockSpec((_NSEG, 2 * d), lambda i: (0, 0)),
        out_shape=jax.ShapeDtypeStruct((_NSEG, 2 * d), jnp.float32),
    )(local, batch3, chain3, mask3)

    t = pl.pallas_call(
        _middle_body,
        out_shape=jax.ShapeDtypeStruct((_NPAIR, d), jnp.float32),
    )(sc, W1, b1.reshape(1, -1), W2, b2.reshape(1, -1))

    out = _sc_broadcast(n, d)(batch_i, chain_i, t)
    return out


# speedup vs baseline: 3.0542x; 2.8813x over previous
import functools

import jax
import jax.numpy as jnp
from jax import lax
from jax.experimental import pallas as pl
from jax.experimental.pallas import tpu as pltpu
from jax.experimental.pallas import tpu_sc as plsc

_NUM_BATCH = 16
_NUM_CHAIN = 128
_NSEG = _NUM_BATCH + _NUM_CHAIN
_NPAIR = _NUM_BATCH * _NUM_CHAIN


def _pick_block(n):
    for r in (3200, 6400, 1600, 800, 400, 320, 160, 80, 40, 16, 8):
        if n % r == 0:
            return r
    return n


def _onehot_bf16(b, c, r):
    seg_iota = lax.broadcasted_iota(jnp.int32, (r, _NSEG), 1)
    return ((b[:, None] == seg_iota).astype(jnp.bfloat16)
            + (c[:, None] == (seg_iota - _NUM_BATCH)).astype(jnp.bfloat16))


def _reduce_body(local_ref, batch_ref, chain_ref, mask_ref, sc_ref):
    @pl.when(pl.program_id(0) == 0)
    def _init():
        sc_ref[...] = jnp.zeros_like(sc_ref)

    x = local_ref[...]
    b = batch_ref[0, 0, :]
    c = chain_ref[0, 0, :]
    m = mask_ref[0, 0, :]
    r, d = x.shape
    onehot = _onehot_bf16(b, c, r) * m[:, None].astype(jnp.bfloat16)
    aug = jnp.concatenate(
        [x.astype(jnp.bfloat16), jnp.ones((r, d), jnp.bfloat16)], axis=1)
    sc_ref[...] += lax.dot_general(
        onehot, aug, (((0,), (0,)), ((), ())),
        preferred_element_type=jnp.float32)


def _middle_body(sc_ref, w1_ref, b1_ref, w2_ref, b2_ref, t_ref):
    d = t_ref.shape[1]
    s = sc_ref[:, :d]
    cnt = sc_ref[:, d:]
    mean = s / jnp.maximum(cnt, 1e-6)
    u = lax.dot_general(
        mean, w1_ref[...], (((1,), (0,)), ((), ())),
        preferred_element_type=jnp.float32,
        precision=lax.Precision.HIGHEST) + b1_ref[0, :]
    g = jax.nn.gelu(u)
    a = lax.dot_general(
        g[:_NUM_BATCH], w2_ref[: u.shape[1]],
        (((1,), (0,)), ((), ())),
        preferred_element_type=jnp.float32,
        precision=lax.Precision.HIGHEST)
    bb = lax.dot_general(
        g[_NUM_BATCH:], w2_ref[u.shape[1]:],
        (((1,), (0,)), ((), ())),
        preferred_element_type=jnp.float32,
        precision=lax.Precision.HIGHEST)
    pair = (a + b2_ref[0, :])[:, None, :] + bb[None, :, :]
    t_ref[...] = pair.reshape(_NPAIR, d)


_CHUNK = 80


def _sc_broadcast(n, d):
    info = plsc.get_sparse_core_info()
    nw = info.num_cores * info.num_subcores
    rows_w = n // nw
    assert n % nw == 0 and rows_w % _CHUNK == 0
    nchunk = rows_w // _CHUNK
    mesh = plsc.VectorSubcoreMesh(core_axis_name="c", subcore_axis_name="s")

    @functools.partial(
        pl.kernel, mesh=mesh,
        out_type=jax.ShapeDtypeStruct((n, d), jnp.float32),
        scratch_types=[
            pltpu.VMEM((rows_w,), jnp.int32),
            pltpu.VMEM((rows_w,), jnp.int32),
            pltpu.VMEM((_CHUNK,), jnp.int32),
            pltpu.VMEM((_CHUNK, d), jnp.float32),
            pltpu.SemaphoreType.DMA,
            pltpu.SemaphoreType.DMA,
        ],
    )
    def k(batch_hbm, chain_hbm, t_hbm, out_hbm, b_v, c_v, p_v, rows_v,
          sem_g, sem_w):
        wid = lax.axis_index("s") * info.num_cores + lax.axis_index("c")
        base = wid * rows_w
        pltpu.sync_copy(batch_hbm.at[pl.ds(base, rows_w)], b_v)
        pltpu.sync_copy(chain_hbm.at[pl.ds(base, rows_w)], c_v)

        def wait_one_write(j, _):
            pltpu.make_async_copy(
                rows_v, out_hbm.at[pl.ds(base, _CHUNK)], sem_w).wait()
            return _

        def chunk_body(i, carry):
            cur_pair, pending = carry
            off = i * _CHUNK
            va_b = b_v[pl.ds(off, 16)]
            vz_b = b_v[pl.ds(off + _CHUNK - 16, 16)]
            va_c = c_v[pl.ds(off, 16)]
            vz_c = c_v[pl.ds(off + _CHUNK - 16, 16)]
            b0, bz = va_b[0], vz_b[15]
            c0, cz = va_c[0], vz_c[15]
            uniform = (b0 == bz) & (c0 == cz)
            pair_now = b0 * _NUM_CHAIN + c0
            reuse = uniform & (pair_now == cur_pair)

            def do_reuse():
                return pending

            def do_gather():
                lax.fori_loop(0, pending, wait_one_write, None)
                for j in range(_CHUNK // 16):
                    sl = pl.ds(off + j * 16, 16)
                    p_v[pl.ds(j * 16, 16)] = b_v[sl] * _NUM_CHAIN + c_v[sl]
                pltpu.async_copy(t_hbm.at[p_v], rows_v, sem_g).wait()
                return jnp.int32(0)

            pending2 = lax.cond(reuse, do_reuse, do_gather)
            pltpu.async_copy(
                rows_v, out_hbm.at[pl.ds(base + off, _CHUNK)], sem_w)
            ui = uniform.astype(jnp.int32)
            new_pair = ui * pair_now + (1 - ui) * jnp.int32(-1)
            return new_pair, pending2 + 1

        _, pending = lax.fori_loop(
            0, nchunk, chunk_body, (jnp.int32(-1), jnp.int32(0)))
        lax.fori_loop(0, pending, wait_one_write, None)

    return k


@jax.jit
def kernel(local, chain, batch, mask, W1, b1, W2, b2):
    n, d = local.shape
    r = _pick_block(n)
    gb = n // r
    batch_i = batch.astype(jnp.int32)
    chain_i = chain.astype(jnp.int32)
    batch3 = batch_i.reshape(gb, 1, r)
    chain3 = chain_i.reshape(gb, 1, r)
    mask3 = mask.astype(jnp.float32).reshape(gb, 1, r)

    sc = pl.pallas_call(
        _reduce_body,
        grid=(gb,),
        in_specs=[
            pl.BlockSpec((r, d), lambda i: (i, 0)),
            pl.BlockSpec((1, 1, r), lambda i: (i, 0, 0)),
            pl.BlockSpec((1, 1, r), lambda i: (i, 0, 0)),
            pl.BlockSpec((1, 1, r), lambda i: (i, 0, 0)),
        ],
        out_specs=pl.BlockSpec((_NSEG, 2 * d), lambda i: (0, 0)),
        out_shape=jax.ShapeDtypeStruct((_NSEG, 2 * d), jnp.float32),
    )(local, batch3, chain3, mask3)

    t = pl.pallas_call(
        _middle_body,
        out_shape=jax.ShapeDtypeStruct((_NPAIR, d), jnp.float32),
    )(sc, W1, b1.reshape(1, -1), W2, b2.reshape(1, -1))

    out = _sc_broadcast(n, d)(batch_i, chain_i, t)
    return out
